# baseline (speedup 1.0000x reference)
import jax
import jax.numpy as jnp
from jax.experimental import pallas as pl
from jax.experimental.pallas import tpu as pltpu

_BF = jnp.bfloat16
_F32 = jnp.float32

_CP = pltpu.CompilerParams(
    dimension_semantics=("parallel",),
    vmem_limit_bytes=58 * 1024 * 1024,
)


def _dot(a, b):
    return jnp.dot(a, b, preferred_element_type=_F32)


def _const_spec(shape):
    return pl.BlockSpec(shape, lambda g: tuple(0 for _ in shape))


def _spatial_body(x_ref, w_ref, b_ref, o_ref):
    y = _dot(x_ref[...], w_ref[...])
    o_ref[...] = jnp.maximum(y + b_ref[...], 0.0).astype(o_ref.dtype)


def _spatial(x2d, w, b, rows):
    M, K = x2d.shape
    Nout = w.shape[1]
    return pl.pallas_call(
        _spatial_body,
        out_shape=jax.ShapeDtypeStruct((M, Nout), _BF),
        grid=(M // rows,),
        in_specs=[pl.BlockSpec((rows, K), lambda g: (g, 0)),
                  _const_spec((K, Nout)), _const_spec((1, Nout))],
        out_specs=pl.BlockSpec((rows, Nout), lambda g: (g, 0)),
        compiler_params=_CP,
    )(x2d, w, b)


def _make_spatial0_body(SB, T, V):
    def _body(x_ref, w_ref, b_ref, o_ref):
        x4 = x_ref[...].astype(_BF)
        Cin = x4.shape[1]
        x2d = jnp.concatenate(
            [x4[:, c].reshape(SB * T, V) for c in range(Cin)], axis=1)
        y = _dot(x2d, w_ref[...])
        o_ref[...] = jnp.maximum(y + b_ref[...], 0.0).astype(o_ref.dtype)

    return _body


def _spatial0(x, w, b, *, SB, T, V):
    Np, C = x.shape[0], x.shape[1]
    Nout = w.shape[1]
    return pl.pallas_call(
        _make_spatial0_body(SB, T, V),
        out_shape=jax.ShapeDtypeStruct((Np * T, Nout), _BF),
        grid=(Np // SB,),
        in_specs=[pl.BlockSpec((SB, C, T, V), lambda g: (g, 0, 0, 0)),
                  _const_spec((V * C, Nout)), _const_spec((1, Nout))],
        out_specs=pl.BlockSpec((SB * T, Nout), lambda g: (g, 0)),
        compiler_params=_CP,
    )(x, w, b)


def _temporal64_acc(y_ref, s, wg0, wg1, w8, TV):
    z = jnp.zeros((128, 64), _BF)
    ys = y_ref[s * TV:(s + 1) * TV, :]
    yp = jnp.concatenate([z, ys, z], axis=0)
    q = jnp.concatenate(
        [yp[0:TV + 128], yp[32:TV + 160],
         yp[64:TV + 192], yp[96:TV + 224]], axis=1)
    acc = _dot(q[0:TV], wg0)
    acc = acc + _dot(q[128:TV + 128], wg1)
    acc = acc + _dot(yp[256:TV + 256], w8)
    return acc


def _make_temporal64_body(res, *, SB, TV):
    def _body(*refs):
        if res:
            y_ref, r_ref, g0_ref, g1_ref, w8_ref, bt_ref, o_ref = refs
        else:
            y_ref, g0_ref, g1_ref, w8_ref, bt_ref, o_ref = refs
        for s in range(SB):
            acc = _temporal64_acc(y_ref, s, g0_ref[...], g1_ref[...],
                                  w8_ref[...], TV)
            if res:
                acc = acc + r_ref[s * TV:(s + 1) * TV, :].astype(_F32)
            acc = acc + bt_ref[...]
            o_ref[s * TV:(s + 1) * TV, :] = jnp.maximum(acc, 0.0).astype(_BF)

    return _body


def _temporal64(y_rows, res_rows, wg0, wg1, w8, bt, *, SB, TV):
    M = y_rows.shape[0]
    rows = SB * TV
    if res_rows is None:
        args = (y_rows, wg0, wg1, w8, bt)
        in_specs = [pl.BlockSpec((rows, 64), lambda g: (g, 0))]
    else:
        args = (y_rows, res_rows, wg0, wg1, w8, bt)
        in_specs = [pl.BlockSpec((rows, 64), lambda g: (g, 0)),
                    pl.BlockSpec((rows, 64), lambda g: (g, 0))]
    in_specs += [_const_spec((256, 64)), _const_spec((256, 64)),
                 _const_spec((64, 64)), _const_spec((1, 64))]
    return pl.pallas_call(
        _make_temporal64_body(res_rows is not None, SB=SB, TV=TV),
        out_shape=jax.ShapeDtypeStruct((M, 64), _BF),
        grid=(M // rows,),
        in_specs=in_specs,
        out_specs=pl.BlockSpec((rows, 64), lambda g: (g, 0)),
        compiler_params=_CP,
    )(*args)


def _make_temporal32_body(*, SB, T, V):
    TV = T * V

    def _body(y_ref, r_ref, g_ref, w8_ref, wr_ref, b_ref, o_ref):
        z = jnp.zeros((128, 32), _BF)
        for s in range(SB):
            ys = y_ref[s * TV:(s + 1) * TV, :]
            yp = jnp.concatenate([z, ys, z], axis=0)
            q = jnp.concatenate([yp[32 * i:32 * i + TV] for i in range(8)],
                                axis=1)
            acc = _dot(q, g_ref[...])
            acc = acc + _dot(yp[256:TV + 256], w8_ref[...])
            acc = acc + _dot(r_ref[s * TV:(s + 1) * TV, :], wr_ref[...])
            acc = acc + b_ref[...]
            o_ref[s] = jnp.maximum(acc, 0.0).T.reshape(32, T, V)

    return _body


def _temporal32(y_rows, res_rows, wg, w8, wr, b, *, SB, T, V):
    TV = T * V
    Np = y_rows.shape[0] // TV
    return pl.pallas_call(
        _make_temporal32_body(SB=SB, T=T, V=V),
        out_shape=jax.ShapeDtypeStruct((Np, 32, T, V), _F32),
        grid=(Np // SB,),
        in_specs=[pl.BlockSpec((SB * TV, 32), lambda g: (g, 0)),
                  pl.BlockSpec((SB * TV, 64), lambda g: (g, 0)),
                  _const_spec((256, 32)), _const_spec((32, 32)),
                  _const_spec((64, 32)), _const_spec((1, 32))],
        out_specs=pl.BlockSpec((SB, 32, T, V), lambda g: (g, 0, 0, 0)),
        compiler_params=_CP,
    )(y_rows, res_rows, wg, w8, wr, b)


def _pack_taps(wt, lo, hi):
    n = hi - lo
    return wt[lo:hi].reshape(n * wt.shape[1], wt.shape[2]).astype(_BF)


def kernel(x,
           l0_wbig, l0_bsp, l0_wt, l0_bt,
           l1_wbig, l1_bsp, l1_wt, l1_bt,
           l2_wbig, l2_bsp, l2_wt, l2_bt, l2_wres, l2_bres):
    N, C, T, V = x.shape
    SB = 4
    if N % SB:
        x = jnp.pad(x, ((0, SB - N % SB), (0, 0), (0, 0), (0, 0)))
    Np = x.shape[0]
    TV = T * V

    w0p = l0_wbig.reshape(V, C, l0_wbig.shape[1]).transpose(1, 0, 2)
    w0p = w0p.reshape(V * C, l0_wbig.shape[1]).astype(_BF)

    y0 = _spatial0(x, w0p, l0_bsp, SB=SB, T=T, V=V)
    t0 = _temporal64(y0.reshape(Np * TV, 64), None,
                     _pack_taps(l0_wt, 0, 4), _pack_taps(l0_wt, 4, 8),
                     l0_wt[8].astype(_BF), l0_bt, SB=SB, TV=TV)

    y1 = _spatial(t0.reshape(Np * T, V * 64), l1_wbig.astype(_BF), l1_bsp,
                  SB * T)
    t1 = _temporal64(y1.reshape(Np * TV, 64), t0,
                     _pack_taps(l1_wt, 0, 4), _pack_taps(l1_wt, 4, 8),
                     l1_wt[8].astype(_BF), l1_bt, SB=SB, TV=TV)

    y2 = _spatial(t1.reshape(Np * T, V * 64), l2_wbig.astype(_BF), l2_bsp,
                  SB * T)
    out = _temporal32(y2.reshape(Np * TV, 32), t1,
                      _pack_taps(l2_wt, 0, 8), l2_wt[8].astype(_BF),
                      l2_wres.astype(_BF), l2_bt + l2_bres, SB=SB, T=T, V=V)

    return out[:N]


# speedup vs baseline: 1.1399x; 1.1399x over previous
import jax
import jax.numpy as jnp
from jax.experimental import pallas as pl
from jax.experimental.pallas import tpu as pltpu

_BF = jnp.bfloat16
_F32 = jnp.float32

_CP = pltpu.CompilerParams(
    dimension_semantics=("parallel",),
    vmem_limit_bytes=58 * 1024 * 1024,
)


def _dot(a, b):
    return jnp.dot(a, b, preferred_element_type=_F32)


def _const_spec(shape):
    return pl.BlockSpec(shape, lambda g: tuple(0 for _ in shape))


def _spatial_body(x_ref, w_ref, b_ref, o_ref):
    y = _dot(x_ref[...], w_ref[...])
    o_ref[...] = jnp.maximum(y + b_ref[...], 0.0).astype(o_ref.dtype)


def _spatial(x2d, w, b, rows):
    M, K = x2d.shape
    Nout = w.shape[1]
    return pl.pallas_call(
        _spatial_body,
        out_shape=jax.ShapeDtypeStruct((M, Nout), _BF),
        grid=(M // rows,),
        in_specs=[pl.BlockSpec((rows, K), lambda g: (g, 0)),
                  _const_spec((K, Nout)), _const_spec((1, Nout))],
        out_specs=pl.BlockSpec((rows, Nout), lambda g: (g, 0)),
        compiler_params=_CP,
    )(x2d, w, b)


def _make_spatial0_body(SB, T, V):
    def _body(x_ref, w_ref, b_ref, o_ref):
        x4 = x_ref[...].astype(_BF)
        Cin = x4.shape[1]
        x2d = jnp.concatenate(
            [x4[:, c].reshape(SB * T, V) for c in range(Cin)], axis=1)
        y = _dot(x2d, w_ref[...])
        o_ref[...] = jnp.maximum(y + b_ref[...], 0.0).astype(o_ref.dtype)

    return _body


def _spatial0(x, w, b, *, SB, T, V):
    Np, C = x.shape[0], x.shape[1]
    Nout = w.shape[1]
    return pl.pallas_call(
        _make_spatial0_body(SB, T, V),
        out_shape=jax.ShapeDtypeStruct((Np * T, Nout), _BF),
        grid=(Np // SB,),
        in_specs=[pl.BlockSpec((SB, C, T, V), lambda g: (g, 0, 0, 0)),
                  _const_spec((V * C, Nout)), _const_spec((1, Nout))],
        out_specs=pl.BlockSpec((SB * T, Nout), lambda g: (g, 0)),
        compiler_params=_CP,
    )(x, w, b)


def _temporal64_acc(y_ref, s, wg0, wg1, w8, TV):
    z = jnp.zeros((128, 64), _BF)
    ys = y_ref[s * TV:(s + 1) * TV, :]
    yp = jnp.concatenate([z, ys, z], axis=0)
    q = jnp.concatenate(
        [yp[0:TV + 128], yp[32:TV + 160],
         yp[64:TV + 192], yp[96:TV + 224]], axis=1)
    acc = _dot(q[0:TV], wg0)
    acc = acc + _dot(q[128:TV + 128], wg1)
    acc = acc + _dot(yp[256:TV + 256], w8)
    return acc


def _make_temporal64_body(res, *, SB, TV):
    def _body(*refs):
        if res:
            y_ref, r_ref, g0_ref, g1_ref, w8_ref, bt_ref, o_ref = refs
        else:
            y_ref, g0_ref, g1_ref, w8_ref, bt_ref, o_ref = refs
        for s in range(SB):
            acc = _temporal64_acc(y_ref, s, g0_ref[...], g1_ref[...],
                                  w8_ref[...], TV)
            if res:
                acc = acc + r_ref[s * TV:(s + 1) * TV, :].astype(_F32)
            acc = acc + bt_ref[...]
            o_ref[s * TV:(s + 1) * TV, :] = jnp.maximum(acc, 0.0).astype(_BF)

    return _body


def _temporal64(y_rows, res_rows, wg0, wg1, w8, bt, *, SB, TV):
    M = y_rows.shape[0]
    rows = SB * TV
    if res_rows is None:
        args = (y_rows, wg0, wg1, w8, bt)
        in_specs = [pl.BlockSpec((rows, 64), lambda g: (g, 0))]
    else:
        args = (y_rows, res_rows, wg0, wg1, w8, bt)
        in_specs = [pl.BlockSpec((rows, 64), lambda g: (g, 0)),
                    pl.BlockSpec((rows, 64), lambda g: (g, 0))]
    in_specs += [_const_spec((256, 64)), _const_spec((256, 64)),
                 _const_spec((64, 64)), _const_spec((1, 64))]
    return pl.pallas_call(
        _make_temporal64_body(res_rows is not None, SB=SB, TV=TV),
        out_shape=jax.ShapeDtypeStruct((M, 64), _BF),
        grid=(M // rows,),
        in_specs=in_specs,
        out_specs=pl.BlockSpec((rows, 64), lambda g: (g, 0)),
        compiler_params=_CP,
    )(*args)


def _make_temporal32_body(*, SB, T, V):
    TV = T * V

    def _body(y_ref, r_ref, g_ref, w8_ref, wr_ref, b_ref, o_ref):
        z = jnp.zeros((128, 32), _BF)
        for s in range(SB):
            ys = y_ref[s * TV:(s + 1) * TV, :]
            yp = jnp.concatenate([z, ys, z], axis=0)
            q = jnp.concatenate([yp[32 * i:32 * i + TV] for i in range(8)],
                                axis=1)
            acc = _dot(q, g_ref[...])
            acc = acc + _dot(yp[256:TV + 256], w8_ref[...])
            acc = acc + _dot(r_ref[s * TV:(s + 1) * TV, :], wr_ref[...])
            acc = acc + b_ref[...]
            o_ref[s * TV:(s + 1) * TV, :] = jnp.maximum(acc, 0.0)

    return _body


def _temporal32(y_rows, res_rows, wg, w8, wr, b, *, SB, T, V):
    TV = T * V
    M = y_rows.shape[0]
    rows = SB * TV
    return pl.pallas_call(
        _make_temporal32_body(SB=SB, T=T, V=V),
        out_shape=jax.ShapeDtypeStruct((M, 32), _F32),
        grid=(M // rows,),
        in_specs=[pl.BlockSpec((rows, 32), lambda g: (g, 0)),
                  pl.BlockSpec((rows, 64), lambda g: (g, 0)),
                  _const_spec((256, 32)), _const_spec((32, 32)),
                  _const_spec((64, 32)), _const_spec((1, 32))],
        out_specs=pl.BlockSpec((rows, 32), lambda g: (g, 0)),
        compiler_params=_CP,
    )(y_rows, res_rows, wg, w8, wr, b)


def _pack_taps(wt, lo, hi):
    n = hi - lo
    return wt[lo:hi].reshape(n * wt.shape[1], wt.shape[2]).astype(_BF)


def kernel(x,
           l0_wbig, l0_bsp, l0_wt, l0_bt,
           l1_wbig, l1_bsp, l1_wt, l1_bt,
           l2_wbig, l2_bsp, l2_wt, l2_bt, l2_wres, l2_bres):
    N, C, T, V = x.shape
    SB = 8
    if N % SB:
        x = jnp.pad(x, ((0, SB - N % SB), (0, 0), (0, 0), (0, 0)))
    Np = x.shape[0]
    TV = T * V

    w0p = l0_wbig.reshape(V, C, l0_wbig.shape[1]).transpose(1, 0, 2)
    w0p = w0p.reshape(V * C, l0_wbig.shape[1]).astype(_BF)

    y0 = _spatial0(x, w0p, l0_bsp, SB=SB, T=T, V=V)
    t0 = _temporal64(y0.reshape(Np * TV, 64), None,
                     _pack_taps(l0_wt, 0, 4), _pack_taps(l0_wt, 4, 8),
                     l0_wt[8].astype(_BF), l0_bt, SB=SB, TV=TV)

    y1 = _spatial(t0.reshape(Np * T, V * 64), l1_wbig.astype(_BF), l1_bsp,
                  SB * T)
    t1 = _temporal64(y1.reshape(Np * TV, 64), t0,
                     _pack_taps(l1_wt, 0, 4), _pack_taps(l1_wt, 4, 8),
                     l1_wt[8].astype(_BF), l1_bt, SB=SB, TV=TV)

    y2 = _spatial(t1.reshape(Np * T, V * 64), l2_wbig.astype(_BF), l2_bsp,
                  SB * T)
    out = _temporal32(y2.reshape(Np * TV, 32), t1,
                      _pack_taps(l2_wt, 0, 8), l2_wt[8].astype(_BF),
                      l2_wres.astype(_BF), l2_bt + l2_bres, SB=SB, T=T, V=V)

    out = out.reshape(Np, T, V, 32)[:N]
    return jnp.transpose(out, (0, 3, 1, 2))
